# baseline (speedup 1.0000x reference)
import functools

import jax
import jax.numpy as jnp
from jax import lax
from jax.experimental import pallas as pl
from jax.experimental.pallas import tpu as pltpu
from jax.experimental.pallas import tpu_sc as plsc

CHUNK = 128


@functools.lru_cache(maxsize=None)
def _make_gather(n_flat: int, dim: int):
    info = plsc.get_sparse_core_info()
    num_cores = info.num_cores
    nw = info.num_cores * info.num_subcores
    assert n_flat % (nw * CHUNK) == 0
    rpw = n_flat // nw
    nchunks = rpw // CHUNK

    mesh = plsc.VectorSubcoreMesh(core_axis_name="c", subcore_axis_name="s")

    @functools.partial(
        pl.kernel,
        mesh=mesh,
        out_type=jax.ShapeDtypeStruct((n_flat, dim), jnp.float32),
        scratch_types=[
            pltpu.VMEM((rpw,), jnp.int32),
            pltpu.VMEM((CHUNK, dim), jnp.float32),
            pltpu.SemaphoreType.DMA,
        ],
        compiler_params=pltpu.CompilerParams(use_tc_tiling_on_sc=False),
    )
    def body(ids_hbm, table_hbm, out_hbm, idx_v, rows_v, sem):
        wid = lax.axis_index("s") * num_cores + lax.axis_index("c")
        base = wid * rpw
        pltpu.sync_copy(ids_hbm.at[pl.ds(base, rpw)], idx_v)

        def step(i, carry):
            off = i * CHUNK
            pltpu.async_copy(
                table_hbm.at[idx_v.at[pl.ds(off, CHUNK)]], rows_v, sem
            ).wait()
            pltpu.sync_copy(rows_v, out_hbm.at[pl.ds(base + off, CHUNK)])
            return carry

        lax.fori_loop(0, nchunks, step, 0)

    return body


def kernel(input_ids, table):
    b, s = input_ids.shape
    flat = input_ids.reshape(b * s).astype(jnp.int32)
    out = _make_gather(b * s, table.shape[1])(flat, table)
    return out.reshape(b, s, table.shape[1])


# speedup vs baseline: 1.1163x; 1.1163x over previous
import functools

import jax
import jax.numpy as jnp
from jax import lax
from jax.experimental import pallas as pl
from jax.experimental.pallas import tpu as pltpu
from jax.experimental.pallas import tpu_sc as plsc

CHUNK = 128
NBUF = 8


@functools.lru_cache(maxsize=None)
def _make_gather(n_flat: int, dim: int):
    info = plsc.get_sparse_core_info()
    num_cores = info.num_cores
    nw = info.num_cores * info.num_subcores
    assert n_flat % (nw * CHUNK * NBUF) == 0
    rpw = n_flat // nw
    nchunks = rpw // CHUNK
    ngroups = nchunks // NBUF

    mesh = plsc.VectorSubcoreMesh(core_axis_name="c", subcore_axis_name="s")

    @functools.partial(
        pl.kernel,
        mesh=mesh,
        out_type=jax.ShapeDtypeStruct((n_flat, dim), jnp.float32),
        scratch_types=[
            pltpu.VMEM((nchunks, CHUNK), jnp.int32),
            [pltpu.VMEM((CHUNK, dim), jnp.float32) for _ in range(NBUF)],
            [pltpu.SemaphoreType.DMA for _ in range(NBUF)],
        ],
        compiler_params=pltpu.CompilerParams(use_tc_tiling_on_sc=False),
    )
    def body(ids_hbm, table_hbm, out_hbm, idx_v, bufs, sems):
        wid = lax.axis_index("s") * num_cores + lax.axis_index("c")
        base = wid * rpw
        pltpu.sync_copy(ids_hbm.at[wid], idx_v)

        def start_gather(i, b):
            pltpu.async_copy(table_hbm.at[idx_v.at[i]], bufs[b], sems[b])

        def wait_gather(i, b):
            pltpu.make_async_copy(
                table_hbm.at[idx_v.at[i]], bufs[b], sems[b]
            ).wait()

        for b in range(NBUF):
            start_gather(b, b)

        def group(j, carry):
            for b in range(NBUF):
                i = j * NBUF + b
                wait_gather(i, b)
                pltpu.sync_copy(bufs[b], out_hbm.at[pl.ds(base + i * CHUNK, CHUNK)])

                @pl.when(j < ngroups - 1)
                def _():
                    start_gather(i + NBUF, b)

            return carry

        lax.fori_loop(0, ngroups, group, 0)

    return body


def kernel(input_ids, table):
    b, s = input_ids.shape
    n_flat = b * s
    info = plsc.get_sparse_core_info()
    nw = info.num_cores * info.num_subcores
    ids = input_ids.reshape(nw, n_flat // (nw * CHUNK), CHUNK).astype(jnp.int32)
    out = _make_gather(n_flat, table.shape[1])(ids, table)
    return out.reshape(b, s, table.shape[1])
